# baseline (speedup 1.0000x reference)
import functools

import jax
import jax.numpy as jnp
from jax import lax
from jax.experimental import pallas as pl
from jax.experimental.pallas import tpu as pltpu
from jax.experimental.pallas import tpu_sc as plsc

S = 1024
N = 50
NMAX = 64
D = 128
T = 3 * N
NC = 2
NS = 16
NW = NC * NS
BG = 64
NG = S // BG
NSLOT = T * NG // NW
NTRIP = NSLOT // 3
PAR = 3
LAG = 1

_mesh = plsc.VectorSubcoreMesh(core_axis_name="c", subcore_axis_name="s")


@functools.partial(
    pl.kernel,
    out_type=jax.ShapeDtypeStruct((T, S, 2 * NMAX + D), jnp.float32),
    mesh=_mesh,
    scratch_types=[
        pltpu.VMEM((NSLOT * BG,), jnp.int32),
        [pltpu.VMEM((3 * BG, D), jnp.float32) for _ in range(PAR)],
        pltpu.VMEM((3, BG, D), jnp.float32),
        [pltpu.SemaphoreType.DMA for _ in range(2 * PAR + 1)],
    ],
)
def _embed_sc(idx, zeros_h, emb, lemb, out, ibuf, dbufs, zbuf, sems):
    wid = lax.axis_index("s") * NC + lax.axis_index("c")
    gsems = sems[0:PAR]
    dsems = sems[PAR:2 * PAR]
    zsem = sems[2 * PAR]
    g = wid // 2
    tbase = (wid % 2) * NSLOT
    bcol = g * BG
    pltpu.sync_copy(idx.at[pl.ds(wid * NSLOT * BG, NSLOT * BG)], ibuf)
    pltpu.sync_copy(zeros_h, zbuf)

    def fire_gathers(k):
        p = k % PAR
        return (
            pltpu.async_copy(emb.at[ibuf.at[pl.ds(3 * BG * k, 2 * BG)]],
                             dbufs[p].at[pl.ds(0, 2 * BG)], gsems[p]),
            pltpu.async_copy(lemb.at[ibuf.at[pl.ds(3 * BG * k + 2 * BG, BG)]],
                             dbufs[p].at[pl.ds(2 * BG, BG)], gsems[p]),
        )

    def fire_scatters(k):
        t0 = tbase + 3 * k
        p = k % PAR
        sds = tuple(
            pltpu.async_copy(dbufs[p].at[pl.ds(r * BG, BG)],
                             out.at[t0 + r, pl.ds(bcol, BG), pl.ds(D, D)],
                             dsems[p])
            for r in range(3))
        zd = pltpu.async_copy(
            zbuf, out.at[pl.ds(t0, 3), pl.ds(bcol, BG), pl.ds(0, D)], zsem)
        return sds, zd

    gds, sds, zds = {}, {}, []
    for i in range(NTRIP + LAG):
        if i < NTRIP:
            if i >= PAR:
                for d in sds[i - PAR]:
                    d.wait()
            gds[i] = fire_gathers(i)
        k = i - LAG
        if k >= 0:
            for d in gds[k]:
                d.wait()
            sd, zd = fire_scatters(k)
            sds[k] = sd
            zds.append(zd)
    for k in range(NTRIP - PAR, NTRIP):
        for d in sds[k]:
            d.wait()
    for d in zds:
        d.wait()


def kernel(examples, labels, embeddings, label_embeddings):
    trip = jnp.stack(
        [examples[:, 0::2], examples[:, 1::2], labels[:, :-1]], axis=2)
    idx = (trip.reshape(S, T)
           .reshape(NG, BG, T)
           .transpose(0, 2, 1)
           .reshape(-1))
    zeros_h = jnp.zeros((3, BG, D), jnp.float32)
    out = _embed_sc(idx, zeros_h, embeddings, label_embeddings)
    return jnp.transpose(out, (1, 0, 2))


# speedup vs baseline: 1.0259x; 1.0259x over previous
import functools

import jax
import jax.numpy as jnp
from jax import lax
from jax.experimental import pallas as pl
from jax.experimental.pallas import tpu as pltpu
from jax.experimental.pallas import tpu_sc as plsc

S = 1024
N = 50
NMAX = 64
D = 128
T = 3 * N
NC = 2
NS = 16
NW = NC * NS
BG = 64
NG = S // BG
NIT = T * NG // NW
PAR = 6
LAG = 2

_mesh = plsc.VectorSubcoreMesh(core_axis_name="c", subcore_axis_name="s")


@functools.partial(
    pl.kernel,
    out_type=jax.ShapeDtypeStruct((T, S, 2 * NMAX + D), jnp.float32),
    mesh=_mesh,
    scratch_types=[
        pltpu.VMEM((NIT * BG,), jnp.int32),
        [pltpu.VMEM((BG, D), jnp.float32) for _ in range(PAR)],
        pltpu.VMEM((BG, D), jnp.float32),
        [pltpu.SemaphoreType.DMA for _ in range(2 * PAR + 1)],
    ],
)
def _embed_sc(idx, zeros_h, emb, lemb, out, ibuf, dbufs, zbuf, sems):
    wid = lax.axis_index("s") * NC + lax.axis_index("c")
    gsems = sems[0:PAR]
    dsems = sems[PAR:2 * PAR]
    zsem = sems[2 * PAR]
    g = wid // 2
    tbase = (wid % 2) * NIT
    bcol = g * BG
    pltpu.sync_copy(idx.at[pl.ds(wid * NIT * BG, NIT * BG)], ibuf)
    pltpu.sync_copy(zeros_h, zbuf)

    def fire_gather(li):
        src = lemb if li % 3 == 2 else emb
        p = li % PAR
        return pltpu.async_copy(
            src.at[ibuf.at[pl.ds(li * BG, BG)]], dbufs[p], gsems[p])

    def fire_scatters(li):
        t = tbase + li
        p = li % PAR
        return (
            pltpu.async_copy(dbufs[p],
                             out.at[t, pl.ds(bcol, BG), pl.ds(D, D)],
                             dsems[p]),
            pltpu.async_copy(zbuf,
                             out.at[t, pl.ds(bcol, BG), pl.ds(0, D)],
                             zsem),
        )

    gds, sds, zds = {}, {}, []
    for i in range(NIT + LAG):
        if i < NIT:
            if i >= PAR:
                sds[i - PAR].wait()
            gds[i] = fire_gather(i)
        k = i - LAG
        if k >= 0:
            gds[k].wait()
            sd, zd = fire_scatters(k)
            sds[k] = sd
            zds.append(zd)
    for k in range(NIT - PAR, NIT):
        sds[k].wait()
    for d in zds:
        d.wait()


def kernel(examples, labels, embeddings, label_embeddings):
    trip = jnp.stack(
        [examples[:, 0::2], examples[:, 1::2], labels[:, :-1]], axis=2)
    idx = (trip.reshape(S, T)
           .reshape(NG, BG, T)
           .transpose(0, 2, 1)
           .reshape(-1))
    zeros_h = jnp.zeros((BG, D), jnp.float32)
    out = _embed_sc(idx, zeros_h, embeddings, label_embeddings)
    return jnp.transpose(out, (1, 0, 2))
